# baseline (speedup 1.0000x reference)
import functools
import math

import jax
import jax.numpy as jnp
from jax import lax
from jax.experimental import pallas as pl
from jax.experimental.pallas import tpu as pltpu
from jax.experimental.pallas import tpu_sc as plsc

NC = 2
NS = 16
NW = NC * NS
LANES = 16

CHUNK = 128
NBUF = 4


@functools.cache
def _build(n_chunks: int, d: int):
    mesh = plsc.VectorSubcoreMesh(core_axis_name="c", subcore_axis_name="s")

    scratch = (
        [pltpu.VMEM((n_chunks, CHUNK), jnp.int32)]
        + [pltpu.VMEM((CHUNK, d), jnp.float32) for _ in range(NBUF)]
        + [pltpu.VMEM((CHUNK, d), jnp.float32) for _ in range(NBUF)]
        + [pltpu.SemaphoreType.DMA for _ in range(2 * NBUF)]
    )

    @functools.partial(
        pl.kernel,
        out_type=jax.ShapeDtypeStruct((NW, n_chunks, CHUNK, d), jnp.float32),
        mesh=mesh,
        scratch_types=scratch,
        compiler_params=pltpu.CompilerParams(use_tc_tiling_on_sc=False),
    )
    def emb(table_hbm, idx_hbm, out_hbm, *s):
        idx_v = s[0]
        gbuf = s[1:1 + NBUF]
        obuf = s[1 + NBUF:1 + 2 * NBUF]
        gsem = s[1 + 2 * NBUF:1 + 3 * NBUF]
        osem = s[1 + 3 * NBUF:1 + 4 * NBUF]

        wid = lax.axis_index("c") * NS + lax.axis_index("s")

        pltpu.sync_copy(idx_hbm.at[wid], idx_v)

        def start_gather(g, b):
            pltpu.make_async_copy(
                table_hbm.at[idx_v.at[g]], gbuf[b], gsem[b]
            ).start()

        def scale(b):
            def row(r, _):
                for c in range(d // LANES):
                    sl = pl.ds(c * LANES, LANES)
                    obuf[b][r, sl] = gbuf[b][r, sl] * 8.0
                return 0

            lax.fori_loop(0, CHUNK, row, 0, unroll=4)

        def step(g, b, wait_out, start_next):
            pltpu.make_async_copy(
                table_hbm.at[idx_v.at[g]], gbuf[b], gsem[b]
            ).wait()
            if wait_out:
                pltpu.make_async_copy(
                    obuf[b], out_hbm.at[wid, g], osem[b]
                ).wait()
            scale(b)
            pltpu.make_async_copy(obuf[b], out_hbm.at[wid, g], osem[b]).start()
            if start_next:
                start_gather(g + NBUF, b)

        n_outer = n_chunks // NBUF

        for b in range(NBUF):
            start_gather(b, b)

        for b in range(NBUF):
            step(b, b, wait_out=False, start_next=True)

        def outer(o, _):
            for b in range(NBUF):
                step(o * NBUF + b, b, wait_out=True, start_next=True)
            return 0

        lax.fori_loop(1, n_outer - 1, outer, 0)

        for b in range(NBUF):
            step((n_outer - 1) * NBUF + b, b, wait_out=True, start_next=False)

        for b in range(NBUF):
            pltpu.make_async_copy(
                obuf[b], out_hbm.at[wid, (n_outer - 1) * NBUF + b], osem[b]
            ).wait()

    return emb


def kernel(x, table):
    d = table.shape[1]
    n = x.size
    assert n % (NW * CHUNK) == 0
    n_chunks = n // (NW * CHUNK)
    idx = x.reshape(NW, n_chunks, CHUNK).astype(jnp.int32)
    out = _build(n_chunks, d)(table, idx)
    return out.reshape(x.shape + (d,))
